# baseline (speedup 1.0000x reference)
import functools

import jax
import jax.numpy as jnp
from jax import lax
from jax.experimental import pallas as pl
from jax.experimental.pallas import tpu as pltpu
from jax.experimental.pallas import tpu_sc as plsc

N = 10000
E = 320000
C = 128
H = C // 2
T = 2
NS = 16
NC = 2
B = 128
EPT = 20480
NB = EPT // B
E_PAD = NS * EPT
ROWS = 20096
RPT = ROWS // NS


NPH = 4
PB = NB // NPH
CB = 2 * B


def _sc_segment_sum(xcat, gix_r, six_r):
    mesh = plsc.VectorSubcoreMesh(core_axis_name="c", subcore_axis_name="s",
                                  num_cores=NC, num_subcores=NS)

    @functools.partial(
        pl.kernel,
        mesh=mesh,
        out_type=jax.ShapeDtypeStruct((NC, ROWS, H), jnp.float32),
        scratch_types=[
            pltpu.VMEM((PB // 2, CB), jnp.int32),
            pltpu.VMEM((PB // 2, CB), jnp.int32),
            pltpu.VMEM((2, CB, H), jnp.float32),
            pltpu.VMEM_SHARED((ROWS, H), jnp.float32),
            pltpu.SemaphoreType.DMA,
            pltpu.SemaphoreType.DMA,
            pltpu.SemaphoreType.DMA,
        ],
        compiler_params=pltpu.CompilerParams(use_tc_tiling_on_sc=False),
    )
    def body(xcat_h, gix_h, six_h, out_h, gix_v, six_v, rows2,
             agg_s, sem_g, sem_s0, sem_s1):
        chunk = [rows2.at[0], rows2.at[1]]
        sem_s = (sem_s0, sem_s1)
        c = lax.axis_index("c")
        s = lax.axis_index("s")

        def drain_scatters(g):
            pltpu.make_async_copy(
                xcat_h.at[pl.ds(0, CB)], chunk[g], sem_s[g]).wait()

        z16 = jnp.zeros((16,), jnp.float32)

        def zvbody(i, _):
            for l in range(H // 16):
                rows2[0, i, pl.ds(l * 16, 16)] = z16
            return 0

        lax.fori_loop(0, CB, zvbody, 0)

        zbase = s * RPT

        def zdbody(k, _):
            start = zbase + jnp.minimum(k * CB, RPT - CB)
            pltpu.sync_copy(chunk[0], agg_s.at[pl.ds(start, CB)])
            return 0

        lax.fori_loop(0, (RPT + CB - 1) // CB, zdbody, 0)

        plsc.subcore_barrier()

        for ph in range(NPH):
            prows = PB // 2
            pltpu.sync_copy(gix_h.at[c, s, pl.ds(ph * prows, prows)], gix_v)
            pltpu.sync_copy(six_h.at[s, pl.ds(ph * prows, prows)], six_v)

            def mainbody(j0, _):
                pltpu.async_copy(
                    xcat_h.at[gix_v.at[j0]], chunk[0], sem_g).wait()
                pltpu.sync_copy(chunk[0], agg_s.at[six_v.at[j0]], add=True)
                return 0

            lax.fori_loop(0, prows, mainbody, 0)

        plsc.subcore_barrier()

        def obody(k, _):
            start = zbase + jnp.minimum(k * CB, RPT - CB)
            pltpu.sync_copy(agg_s.at[pl.ds(start, CB)],
                            out_h.at[c, pl.ds(start, CB)])
            return 0

        lax.fori_loop(0, (RPT + CB - 1) // CB, obody, 0)

    return body(xcat, gix_r, six_r)


def _tc_dense(x, agg, weight, rw, bias):
    BLK = 1000
    nbk = N // BLK

    def body(x_b, a00, a01, a10, a11, w, rw_b, b_b, o):
        acc = jnp.dot(x_b[...], rw_b[...], preferred_element_type=jnp.float32)
        acc += jnp.dot(a00[0], w[0, :H, :], preferred_element_type=jnp.float32)
        acc += jnp.dot(a10[0], w[0, H:, :], preferred_element_type=jnp.float32)
        acc += jnp.dot(a01[0], w[1, :H, :], preferred_element_type=jnp.float32)
        acc += jnp.dot(a11[0], w[1, H:, :], preferred_element_type=jnp.float32)
        o[...] = acc + b_b[...]

    def agg_spec(t, h):
        return pl.BlockSpec((1, BLK, H),
                            lambda i, _t=t, _h=h: (_h, i + _t * nbk, 0))

    return pl.pallas_call(
        body,
        grid=(nbk,),
        in_specs=[
            pl.BlockSpec((BLK, C), lambda i: (i, 0)),
            agg_spec(0, 0),
            agg_spec(1, 0),
            agg_spec(0, 1),
            agg_spec(1, 1),
            pl.BlockSpec((T, C, C), lambda i: (0, 0, 0)),
            pl.BlockSpec((C, C), lambda i: (0, 0)),
            pl.BlockSpec((1, C), lambda i: (0, 0)),
        ],
        out_specs=pl.BlockSpec((BLK, C), lambda i: (i, 0)),
        out_shape=jax.ShapeDtypeStruct((N, C), jnp.float32),
    )(x, agg, agg, agg, agg, weight, rw, bias)


@jax.jit
def kernel(x, edge_index, edge_type, weight, root_w, root_b):
    src = edge_index[0]
    dst = edge_index[1]

    xcat = jnp.concatenate([x[:, :H], x[:, H:]], axis=0)
    pad = E_PAD - E
    src_p = jnp.concatenate([src, jnp.zeros((pad,), jnp.int32)])
    gix_r = jnp.stack([src_p, src_p + N]).reshape(NC, NS, EPT // CB, CB)
    six = edge_type * N + dst
    six_r = jnp.concatenate([six, jnp.full((pad,), T * N, jnp.int32)]).reshape(
        NS, EPT // CB, CB)

    agg = _sc_segment_sum(xcat, gix_r, six_r)
    return _tc_dense(x, agg, weight, root_w.T, root_b.reshape(1, C))


# speedup vs baseline: 1.1200x; 1.1200x over previous
import functools

import jax
import jax.numpy as jnp
from jax import lax
from jax.experimental import pallas as pl
from jax.experimental.pallas import tpu as pltpu
from jax.experimental.pallas import tpu_sc as plsc

N = 10000
E = 320000
C = 128
H = C // 2
T = 2
NS = 16
NC = 2
B = 128
EPT = 20480
NB = EPT // B
E_PAD = NS * EPT
ROWS = 20096
RPT = ROWS // NS


NPH = 4
PB = NB // NPH
CB = 2 * B


def _sc_segment_sum(xcat, gix_r, six_r):
    mesh = plsc.VectorSubcoreMesh(core_axis_name="c", subcore_axis_name="s",
                                  num_cores=NC, num_subcores=NS)

    @functools.partial(
        pl.kernel,
        mesh=mesh,
        out_type=jax.ShapeDtypeStruct((NC, ROWS, H), jnp.float32),
        scratch_types=[
            pltpu.VMEM((PB, B), jnp.int32),
            pltpu.VMEM((PB, B), jnp.int32),
            pltpu.VMEM((4, B, H), jnp.float32),
            pltpu.VMEM_SHARED((ROWS, H), jnp.float32),
            pltpu.SemaphoreType.DMA,
        ],
        compiler_params=pltpu.CompilerParams(use_tc_tiling_on_sc=False),
    )
    def body(xcat_h, gix_h, six_h, out_h, gix_v, six_v, rows4,
             agg_s, sem_g):
        bufs = [rows4.at[i] for i in range(4)]
        c = lax.axis_index("c")
        s = lax.axis_index("s")

        z16 = jnp.zeros((16,), jnp.float32)

        def zvbody(i, _):
            for l in range(H // 16):
                rows4[0, i, pl.ds(l * 16, 16)] = z16
            return 0

        lax.fori_loop(0, B, zvbody, 0)

        zbase = s * RPT

        def zdbody(k, _):
            start = zbase + jnp.minimum(k * B, RPT - B)
            pltpu.sync_copy(bufs[0], agg_s.at[pl.ds(start, B)])
            return 0

        lax.fori_loop(0, (RPT + B - 1) // B, zdbody, 0)

        plsc.subcore_barrier()

        NR = PB // 4
        for ph in range(NPH):
            pltpu.sync_copy(gix_h.at[c, s, pl.ds(ph * PB, PB)], gix_v)
            pltpu.sync_copy(six_h.at[s, pl.ds(ph * PB, PB)], six_v)

            for b in range(4):
                pltpu.async_copy(xcat_h.at[gix_v.at[b]], bufs[b], sem_g)

            def mainbody(kk, _):
                for b in range(4):
                    pltpu.make_async_copy(
                        xcat_h.at[pl.ds(0, B)], bufs[b], sem_g).wait()
                    pltpu.sync_copy(
                        bufs[b], agg_s.at[six_v.at[kk * 4 + b]], add=True)

                @pl.when(kk < NR - 1)
                def _():
                    for b in range(4):
                        pltpu.async_copy(
                            xcat_h.at[gix_v.at[(kk + 1) * 4 + b]], bufs[b],
                            sem_g)
                return 0

            lax.fori_loop(0, NR, mainbody, 0)

        plsc.subcore_barrier()

        def obody(k, _):
            start = zbase + jnp.minimum(k * CB, RPT - CB)
            pltpu.sync_copy(agg_s.at[pl.ds(start, CB)],
                            out_h.at[c, pl.ds(start, CB)])
            return 0

        lax.fori_loop(0, (RPT + CB - 1) // CB, obody, 0)

    return body(xcat, gix_r, six_r)


def _tc_dense(x, agg, weight, rw, bias):
    BLK = 1000
    nbk = N // BLK

    def body(x_b, a00, a01, a10, a11, w, rw_b, b_b, o):
        acc = jnp.dot(x_b[...], rw_b[...], preferred_element_type=jnp.float32)
        acc += jnp.dot(a00[0], w[0, :H, :], preferred_element_type=jnp.float32)
        acc += jnp.dot(a10[0], w[0, H:, :], preferred_element_type=jnp.float32)
        acc += jnp.dot(a01[0], w[1, :H, :], preferred_element_type=jnp.float32)
        acc += jnp.dot(a11[0], w[1, H:, :], preferred_element_type=jnp.float32)
        o[...] = acc + b_b[...]

    def agg_spec(t, h):
        return pl.BlockSpec((1, BLK, H),
                            lambda i, _t=t, _h=h: (_h, i + _t * nbk, 0))

    return pl.pallas_call(
        body,
        grid=(nbk,),
        in_specs=[
            pl.BlockSpec((BLK, C), lambda i: (i, 0)),
            agg_spec(0, 0),
            agg_spec(1, 0),
            agg_spec(0, 1),
            agg_spec(1, 1),
            pl.BlockSpec((T, C, C), lambda i: (0, 0, 0)),
            pl.BlockSpec((C, C), lambda i: (0, 0)),
            pl.BlockSpec((1, C), lambda i: (0, 0)),
        ],
        out_specs=pl.BlockSpec((BLK, C), lambda i: (i, 0)),
        out_shape=jax.ShapeDtypeStruct((N, C), jnp.float32),
    )(x, agg, agg, agg, agg, weight, rw, bias)


@jax.jit
def kernel(x, edge_index, edge_type, weight, root_w, root_b):
    src = edge_index[0]
    dst = edge_index[1]

    xcat = jnp.concatenate([x[:, :H], x[:, H:]], axis=0)
    pad = E_PAD - E
    src_p = jnp.concatenate([src, jnp.zeros((pad,), jnp.int32)])
    gix_r = jnp.stack([src_p, src_p + N]).reshape(NC, NS, NB, B)
    six = edge_type * N + dst
    six_r = jnp.concatenate([six, jnp.full((pad,), T * N, jnp.int32)]).reshape(
        NS, NB, B)

    agg = _sc_segment_sum(xcat, gix_r, six_r)
    return _tc_dense(x, agg, weight, root_w.T, root_b.reshape(1, C))


# speedup vs baseline: 1.1246x; 1.0041x over previous
import functools

import jax
import jax.numpy as jnp
from jax import lax
from jax.experimental import pallas as pl
from jax.experimental.pallas import tpu as pltpu
from jax.experimental.pallas import tpu_sc as plsc

N = 10000
E = 320000
C = 128
H = C // 2
T = 2
NS = 16
NC = 2
B = 128
EPT = 20480
NB = EPT // B
E_PAD = NS * EPT
ROWS = 20096
RPT = ROWS // NS


NPH = 4
PB = NB // NPH
CB = 2 * B


def _sc_segment_sum(xcat, gix_r, six_r):
    mesh = plsc.VectorSubcoreMesh(core_axis_name="c", subcore_axis_name="s",
                                  num_cores=NC, num_subcores=NS)

    @functools.partial(
        pl.kernel,
        mesh=mesh,
        out_type=jax.ShapeDtypeStruct((NC, ROWS, H), jnp.float32),
        scratch_types=[
            pltpu.VMEM((PB, B), jnp.int32),
            pltpu.VMEM((PB, B), jnp.int32),
            pltpu.VMEM((4, B, H), jnp.float32),
            pltpu.VMEM_SHARED((ROWS, H), jnp.float32),
            pltpu.SemaphoreType.DMA,
            pltpu.SemaphoreType.DMA,
        ],
        compiler_params=pltpu.CompilerParams(use_tc_tiling_on_sc=False),
    )
    def body(xcat_h, gix_h, six_h, out_h, gix_v, six_v, rows4,
             agg_s, sem_g, sem_s):
        bufs = [rows4.at[i] for i in range(4)]
        c = lax.axis_index("c")
        s = lax.axis_index("s")

        z16 = jnp.zeros((16,), jnp.float32)

        def zvbody(i, _):
            for l in range(H // 16):
                rows4[0, i, pl.ds(l * 16, 16)] = z16
            return 0

        lax.fori_loop(0, B, zvbody, 0)

        zbase = s * RPT

        def zdbody(k, _):
            start = zbase + jnp.minimum(k * B, RPT - B)
            pltpu.sync_copy(bufs[0], agg_s.at[pl.ds(start, B)])
            return 0

        lax.fori_loop(0, (RPT + B - 1) // B, zdbody, 0)

        plsc.subcore_barrier()

        for ph in range(NPH):
            pltpu.sync_copy(gix_h.at[c, s, pl.ds(ph * PB, PB)], gix_v)
            pltpu.sync_copy(six_h.at[s, pl.ds(ph * PB, PB)], six_v)

            def mainbody(kk, _):
                gds = [
                    pltpu.async_copy(
                        xcat_h.at[gix_v.at[kk * 4 + b]], bufs[b], sem_g)
                    for b in range(4)
                ]
                sds = []
                for b in range(4):
                    gds[b].wait()
                    sds.append(pltpu.async_copy(
                        bufs[b], agg_s.at[six_v.at[kk * 4 + b]], sem_s,
                        add=True))
                for d in sds:
                    d.wait()
                return 0

            lax.fori_loop(0, PB // 4, mainbody, 0)

        plsc.subcore_barrier()

        def obody(k, _):
            start = zbase + jnp.minimum(k * CB, RPT - CB)
            pltpu.sync_copy(agg_s.at[pl.ds(start, CB)],
                            out_h.at[c, pl.ds(start, CB)])
            return 0

        lax.fori_loop(0, (RPT + CB - 1) // CB, obody, 0)

    return body(xcat, gix_r, six_r)


def _tc_dense(x, agg, weight, rw, bias):
    BLK = 1000
    nbk = N // BLK

    def body(x_b, a00, a01, a10, a11, w, rw_b, b_b, o):
        acc = jnp.dot(x_b[...], rw_b[...], preferred_element_type=jnp.float32)
        acc += jnp.dot(a00[0], w[0, :H, :], preferred_element_type=jnp.float32)
        acc += jnp.dot(a10[0], w[0, H:, :], preferred_element_type=jnp.float32)
        acc += jnp.dot(a01[0], w[1, :H, :], preferred_element_type=jnp.float32)
        acc += jnp.dot(a11[0], w[1, H:, :], preferred_element_type=jnp.float32)
        o[...] = acc + b_b[...]

    def agg_spec(t, h):
        return pl.BlockSpec((1, BLK, H),
                            lambda i, _t=t, _h=h: (_h, i + _t * nbk, 0))

    return pl.pallas_call(
        body,
        grid=(nbk,),
        in_specs=[
            pl.BlockSpec((BLK, C), lambda i: (i, 0)),
            agg_spec(0, 0),
            agg_spec(1, 0),
            agg_spec(0, 1),
            agg_spec(1, 1),
            pl.BlockSpec((T, C, C), lambda i: (0, 0, 0)),
            pl.BlockSpec((C, C), lambda i: (0, 0)),
            pl.BlockSpec((1, C), lambda i: (0, 0)),
        ],
        out_specs=pl.BlockSpec((BLK, C), lambda i: (i, 0)),
        out_shape=jax.ShapeDtypeStruct((N, C), jnp.float32),
    )(x, agg, agg, agg, agg, weight, rw, bias)


@jax.jit
def kernel(x, edge_index, edge_type, weight, root_w, root_b):
    src = edge_index[0]
    dst = edge_index[1]

    xcat = jnp.concatenate([x[:, :H], x[:, H:]], axis=0)
    pad = E_PAD - E
    src_p = jnp.concatenate([src, jnp.zeros((pad,), jnp.int32)])
    gix_r = jnp.stack([src_p, src_p + N]).reshape(NC, NS, NB, B)
    six = edge_type * N + dst
    six_r = jnp.concatenate([six, jnp.full((pad,), T * N, jnp.int32)]).reshape(
        NS, NB, B)

    agg = _sc_segment_sum(xcat, gix_r, six_r)
    return _tc_dense(x, agg, weight, root_w.T, root_b.reshape(1, C))
